# Initial kernel scaffold; baseline (speedup 1.0000x reference)
import jax
import jax.numpy as jnp
from jax.experimental import pallas as pl


def kernel(query, pool, keys, W_out):
    raise NotImplementedError("write your pallas kernel here")


# speedup vs baseline: 25.6633x; 25.6633x over previous
import functools

import jax
import jax.numpy as jnp
from jax.experimental import pallas as pl
from jax.experimental.pallas import tpu as pltpu

_POOL = 100000
_D = 256
_K = 32
_ROWS = 2048
_SUB = 32
_CB = 4096
_NCHUNK = 25
_NPAD = _CB * _NCHUNK
_NSUB = _NPAD // _SUB
_RB = 256
_NEG = float("-inf")


_ARB = 256
_NSUB_REAL = _POOL // _SUB


def _score_body(q_ref, k_ref, s_ref, m_ref):
    c = pl.program_id(0)
    s = jax.lax.dot_general(
        q_ref[...], k_ref[...], (((1,), (1,)), ((), ())),
        preferred_element_type=jnp.float32)
    s_ref[...] = s
    st = jax.lax.dot_general(
        k_ref[...], q_ref[...], (((1,), (1,)), ((), ())),
        preferred_element_type=jnp.float32)
    m = jnp.max(st.reshape(_CB // _SUB, _SUB, _ARB), axis=1)
    sub = jax.lax.broadcasted_iota(jnp.int32, m.shape, 0) + c * (_CB // _SUB)
    m_ref[...] = jnp.where(sub < _NSUB_REAL, m, _NEG)


def _topk_body(x_ref, idx_ref, scratch):
    scratch[...] = x_ref[...]
    n, r = x_ref.shape
    row = jax.lax.broadcasted_iota(jnp.int32, (n, r), 0)
    slot = jax.lax.broadcasted_iota(jnp.int32, (_K, r), 0)

    def step(j, idxs):
        x = scratch[...]
        m = jnp.max(x, axis=0, keepdims=True)
        idx = jnp.min(jnp.where(x == m, row, n), axis=0, keepdims=True)
        idxs = jnp.where(slot == j, idx, idxs)
        scratch[...] = jnp.where(row == idx, _NEG, x)
        return idxs

    idx_ref[...] = jax.lax.fori_loop(
        0, _K, step, jnp.zeros((_K, r), jnp.int32))


def _select_body(cand_ref, sub_ref, w_ref, gidx_ref, scratch):
    scratch[...] = cand_ref[...]
    r, n = cand_ref.shape
    col = jax.lax.broadcasted_iota(jnp.int32, (r, n), 1)
    lane32 = jax.lax.broadcasted_iota(jnp.int32, (r, _K), 1)
    subs = sub_ref[...]

    def step(j, carry):
        vals, gidx = carry
        x = scratch[...]
        m = jnp.max(x, axis=1, keepdims=True)
        idx = jnp.min(jnp.where(x == m, col, n), axis=1, keepdims=True)
        sub_slot = idx // _SUB
        sub_val = jnp.sum(
            jnp.where(lane32 == sub_slot, subs, 0),
            axis=1, keepdims=True)
        vals = jnp.where(lane32 == j, m, vals)
        gidx = jnp.where(lane32 == j, sub_val * _SUB + idx % _SUB, gidx)
        scratch[...] = jnp.where(col == idx, _NEG, x)
        return vals, gidx

    vals, gidx = jax.lax.fori_loop(
        0, _K, step,
        (jnp.zeros((r, _K), jnp.float32), jnp.zeros((r, _K), jnp.int32)))
    gidx_ref[...] = gidx
    m = jnp.max(vals, axis=1, keepdims=True)
    e = jnp.exp(vals - m)
    w_ref[...] = e / jnp.sum(e, axis=1, keepdims=True)


def _agg_body(g_ref, w_ref, wout_ref, o_ref):
    w = w_ref[...]
    acc = jnp.zeros((_RB, _D), jnp.float32)
    for k in range(_K):
        acc = acc + g_ref[:, k, :] * w[:, k:k + 1]
    o_ref[...] = jax.lax.dot_general(
        acc, wout_ref[...], (((1,), (1,)), ((), ())),
        preferred_element_type=jnp.float32)


@jax.jit
def kernel(query, pool, keys, W_out):
    q2 = query.reshape(_ROWS, _D)

    scores, maxes = pl.pallas_call(
        _score_body,
        grid=(_NCHUNK, _ROWS // _ARB),
        in_specs=[
            pl.BlockSpec((_ARB, _D), lambda c, r: (r, 0)),
            pl.BlockSpec((_CB, _D), lambda c, r: (c, 0)),
        ],
        out_specs=[
            pl.BlockSpec((_ARB, _CB), lambda c, r: (r, c)),
            pl.BlockSpec((_CB // _SUB, _ARB), lambda c, r: (c, r)),
        ],
        out_shape=[
            jax.ShapeDtypeStruct((_ROWS, _NPAD), jnp.float32),
            jax.ShapeDtypeStruct((_NSUB, _ROWS), jnp.float32),
        ],
    )(q2, keys)

    sub_ids_t = pl.pallas_call(
        _topk_body,
        grid=(_ROWS // _RB,),
        in_specs=[pl.BlockSpec((_NSUB, _RB), lambda r: (0, r))],
        out_specs=pl.BlockSpec((_K, _RB), lambda r: (0, r)),
        out_shape=jax.ShapeDtypeStruct((_K, _ROWS), jnp.int32),
        scratch_shapes=[pltpu.VMEM((_NSUB, _RB), jnp.float32)],
    )(maxes)
    sub_ids = sub_ids_t.T

    row = jnp.arange(_ROWS, dtype=jnp.int32)[:, None]
    flat = (row * _NSUB + sub_ids).reshape(-1)
    cand = jnp.take(scores.reshape(_ROWS * _NSUB, _SUB), flat, axis=0)
    cand = cand.reshape(_ROWS, _K * _SUB)

    weights, gidx = pl.pallas_call(
        _select_body,
        grid=(_ROWS // _RB,),
        in_specs=[
            pl.BlockSpec((_RB, _K * _SUB), lambda r: (r, 0)),
            pl.BlockSpec((_RB, _K), lambda r: (r, 0)),
        ],
        out_specs=[
            pl.BlockSpec((_RB, _K), lambda r: (r, 0)),
            pl.BlockSpec((_RB, _K), lambda r: (r, 0)),
        ],
        out_shape=[
            jax.ShapeDtypeStruct((_ROWS, _K), jnp.float32),
            jax.ShapeDtypeStruct((_ROWS, _K), jnp.int32),
        ],
        scratch_shapes=[pltpu.VMEM((_RB, _K * _SUB), jnp.float32)],
    )(cand, sub_ids)

    gathered = jnp.take(pool, gidx.reshape(-1), axis=0)
    gathered = gathered.reshape(_ROWS, _K, _D)

    out = pl.pallas_call(
        _agg_body,
        grid=(_ROWS // _RB,),
        in_specs=[
            pl.BlockSpec((_RB, _K, _D), lambda r: (r, 0, 0)),
            pl.BlockSpec((_RB, _K), lambda r: (r, 0)),
            pl.BlockSpec((_D, _D), lambda r: (0, 0)),
        ],
        out_specs=pl.BlockSpec((_RB, _D), lambda r: (r, 0)),
        out_shape=jax.ShapeDtypeStruct((_ROWS, _D), jnp.float32),
    )(gathered, weights, W_out)

    return out.reshape(query.shape[0], query.shape[1], _D)


# speedup vs baseline: 87.7146x; 3.4179x over previous
import functools

import jax
import jax.numpy as jnp
from jax import lax
from jax.experimental import pallas as pl
from jax.experimental.pallas import tpu as pltpu
from jax.experimental.pallas import tpu_sc as plsc

_POOL = 100000
_D = 256
_K = 32
_ROWS = 2048
_SUB = 32
_CB = 4096
_NCHUNK = 25
_NPAD = _CB * _NCHUNK
_NSUB = _NPAD // _SUB
_RB = 256
_NEG = float("-inf")


_ARB = 256
_NSUB_REAL = _POOL // _SUB


def _score_body(q_ref, k_ref, s_ref, m_ref):
    c = pl.program_id(0)
    s = jax.lax.dot_general(
        q_ref[...], k_ref[...], (((1,), (1,)), ((), ())),
        preferred_element_type=jnp.float32)
    s_ref[...] = s
    st = jax.lax.dot_general(
        k_ref[...], q_ref[...], (((1,), (1,)), ((), ())),
        preferred_element_type=jnp.float32)
    m = jnp.max(st.reshape(_CB // _SUB, _SUB, _ARB), axis=1)
    sub = jax.lax.broadcasted_iota(jnp.int32, m.shape, 0) + c * (_CB // _SUB)
    m_ref[...] = jnp.where(sub < _NSUB_REAL, m, _NEG)


def _topk_body(x_ref, idx_ref, scratch):
    scratch[...] = x_ref[...]
    n, r = x_ref.shape
    row = jax.lax.broadcasted_iota(jnp.int32, (n, r), 0)
    slot = jax.lax.broadcasted_iota(jnp.int32, (_K, r), 0)

    def step(j, idxs):
        x = scratch[...]
        m = jnp.max(x, axis=0, keepdims=True)
        idx = jnp.min(jnp.where(x == m, row, n), axis=0, keepdims=True)
        idxs = jnp.where(slot == j, idx, idxs)
        scratch[...] = jnp.where(row == idx, _NEG, x)
        return idxs

    idx_ref[...] = jax.lax.fori_loop(
        0, _K, step, jnp.zeros((_K, r), jnp.int32))


def _select_body(cand_ref, sub_ref, w_ref, gidx_ref, scratch):
    scratch[...] = cand_ref[...]
    r, n = cand_ref.shape
    col = jax.lax.broadcasted_iota(jnp.int32, (r, n), 1)
    lane32 = jax.lax.broadcasted_iota(jnp.int32, (r, _K), 1)
    subs = sub_ref[...]

    def step(j, carry):
        vals, gidx = carry
        x = scratch[...]
        m = jnp.max(x, axis=1, keepdims=True)
        idx = jnp.min(jnp.where(x == m, col, n), axis=1, keepdims=True)
        sub_slot = idx // _SUB
        sub_val = jnp.sum(
            jnp.where(lane32 == sub_slot, subs, 0),
            axis=1, keepdims=True)
        vals = jnp.where(lane32 == j, m, vals)
        gidx = jnp.where(lane32 == j, sub_val * _SUB + idx % _SUB, gidx)
        scratch[...] = jnp.where(col == idx, _NEG, x)
        return vals, gidx

    vals, gidx = jax.lax.fori_loop(
        0, _K, step,
        (jnp.zeros((r, _K), jnp.float32), jnp.zeros((r, _K), jnp.int32)))
    gidx_ref[...] = gidx
    m = jnp.max(vals, axis=1, keepdims=True)
    e = jnp.exp(vals - m)
    w_ref[...] = e / jnp.sum(e, axis=1, keepdims=True)


_NC = 2
_NS = 16
_NW = _NC * _NS


def _sc_gather(table, idx2d, d):
    n_chunks = idx2d.shape[0]
    per_w = n_chunks // _NW
    mesh = plsc.VectorSubcoreMesh(core_axis_name="c", subcore_axis_name="s")

    @functools.partial(
        pl.kernel,
        out_type=jax.ShapeDtypeStruct((n_chunks * 128, d), jnp.float32),
        mesh=mesh,
        compiler_params=pltpu.CompilerParams(use_tc_tiling_on_sc=False),
        scratch_types=[
            pltpu.VMEM((per_w, 128), jnp.int32),
            pltpu.VMEM((128, d), jnp.float32),
            pltpu.SemaphoreType.DMA,
        ],
    )
    def k(table_hbm, idx_hbm, out_hbm, idx_v, rows_v, sem):
        wid = lax.axis_index("s") * _NC + lax.axis_index("c")
        pltpu.sync_copy(idx_hbm.at[pl.ds(wid * per_w, per_w)], idx_v)
        for j in range(per_w):
            pltpu.async_copy(table_hbm.at[idx_v.at[j]], rows_v, sem).wait()
            pltpu.sync_copy(
                rows_v, out_hbm.at[pl.ds((wid * per_w + j) * 128, 128)])

    return k(table, idx2d)


def _agg_body(g_ref, w_ref, wout_ref, o_ref):
    w = w_ref[...]
    acc = jnp.zeros((_RB, _D), jnp.float32)
    for k in range(_K):
        acc = acc + g_ref[:, k, :] * w[:, k:k + 1]
    o_ref[...] = jax.lax.dot_general(
        acc, wout_ref[...], (((1,), (1,)), ((), ())),
        preferred_element_type=jnp.float32)


@jax.jit
def kernel(query, pool, keys, W_out):
    q2 = query.reshape(_ROWS, _D)

    scores, maxes = pl.pallas_call(
        _score_body,
        grid=(_NCHUNK, _ROWS // _ARB),
        in_specs=[
            pl.BlockSpec((_ARB, _D), lambda c, r: (r, 0)),
            pl.BlockSpec((_CB, _D), lambda c, r: (c, 0)),
        ],
        out_specs=[
            pl.BlockSpec((_ARB, _CB), lambda c, r: (r, c)),
            pl.BlockSpec((_CB // _SUB, _ARB), lambda c, r: (c, r)),
        ],
        out_shape=[
            jax.ShapeDtypeStruct((_ROWS, _NPAD), jnp.float32),
            jax.ShapeDtypeStruct((_NSUB, _ROWS), jnp.float32),
        ],
    )(q2, keys)

    sub_ids_t = pl.pallas_call(
        _topk_body,
        grid=(_ROWS // _RB,),
        in_specs=[pl.BlockSpec((_NSUB, _RB), lambda r: (0, r))],
        out_specs=pl.BlockSpec((_K, _RB), lambda r: (0, r)),
        out_shape=jax.ShapeDtypeStruct((_K, _ROWS), jnp.int32),
        scratch_shapes=[pltpu.VMEM((_NSUB, _RB), jnp.float32)],
    )(maxes)
    sub_ids = sub_ids_t.T

    row = jnp.arange(_ROWS, dtype=jnp.int32)[:, None]
    flat = (row * _NSUB + sub_ids).reshape(-1, 128)
    cand = _sc_gather(scores.reshape(_ROWS * _NSUB, _SUB), flat, _SUB)
    cand = cand.reshape(_ROWS, _K * _SUB)

    weights, gidx = pl.pallas_call(
        _select_body,
        grid=(_ROWS // _RB,),
        in_specs=[
            pl.BlockSpec((_RB, _K * _SUB), lambda r: (r, 0)),
            pl.BlockSpec((_RB, _K), lambda r: (r, 0)),
        ],
        out_specs=[
            pl.BlockSpec((_RB, _K), lambda r: (r, 0)),
            pl.BlockSpec((_RB, _K), lambda r: (r, 0)),
        ],
        out_shape=[
            jax.ShapeDtypeStruct((_ROWS, _K), jnp.float32),
            jax.ShapeDtypeStruct((_ROWS, _K), jnp.int32),
        ],
        scratch_shapes=[pltpu.VMEM((_RB, _K * _SUB), jnp.float32)],
    )(cand, sub_ids)

    gathered = _sc_gather(pool, gidx.reshape(-1, 128), _D)
    gathered = gathered.reshape(_ROWS, _K, _D)

    out = pl.pallas_call(
        _agg_body,
        grid=(_ROWS // _RB,),
        in_specs=[
            pl.BlockSpec((_RB, _K, _D), lambda r: (r, 0, 0)),
            pl.BlockSpec((_RB, _K), lambda r: (r, 0)),
            pl.BlockSpec((_D, _D), lambda r: (0, 0)),
        ],
        out_specs=pl.BlockSpec((_RB, _D), lambda r: (r, 0)),
        out_shape=jax.ShapeDtypeStruct((_ROWS, _D), jnp.float32),
    )(gathered, weights, W_out)

    return out.reshape(query.shape[0], query.shape[1], _D)


# speedup vs baseline: 103.2013x; 1.1766x over previous
import functools

import jax
import jax.numpy as jnp
from jax import lax
from jax.experimental import pallas as pl
from jax.experimental.pallas import tpu as pltpu
from jax.experimental.pallas import tpu_sc as plsc

_POOL = 100000
_D = 256
_K = 32
_ROWS = 2048
_SUB = 32
_CB = 4096
_NCHUNK = 25
_NPAD = _CB * _NCHUNK
_NSUB = _NPAD // _SUB
_RB = 256
_NEG = float("-inf")


_ARB = 256
_NSUB_REAL = _POOL // _SUB


def _score_body(q_ref, k_ref, s_ref, m_ref):
    c = pl.program_id(0)
    s = jax.lax.dot_general(
        q_ref[...], k_ref[...], (((1,), (1,)), ((), ())),
        preferred_element_type=jnp.float32)
    s_ref[...] = s.reshape(_ARB, _CB // 128, 128)
    st = jax.lax.dot_general(
        k_ref[...], q_ref[...], (((1,), (1,)), ((), ())),
        preferred_element_type=jnp.float32)
    m = jnp.max(st.reshape(_CB // _SUB, _SUB, _ARB), axis=1)
    sub = jax.lax.broadcasted_iota(jnp.int32, m.shape, 0) + c * (_CB // _SUB)
    m_ref[...] = jnp.where(sub < _NSUB_REAL, m, _NEG)


def _topk_body(x_ref, idx_ref, scratch):
    scratch[...] = x_ref[...]
    n, r = x_ref.shape
    row = jax.lax.broadcasted_iota(jnp.int32, (n, r), 0)
    slot = jax.lax.broadcasted_iota(jnp.int32, (_K, r), 0)

    def step(j, idxs):
        x = scratch[...]
        m = jnp.max(x, axis=0, keepdims=True)
        idx = jnp.min(jnp.where(x == m, row, n), axis=0, keepdims=True)
        idxs = jnp.where(slot == j, idx, idxs)
        scratch[...] = jnp.where(row == idx, _NEG, x)
        return idxs

    idx_ref[...] = jax.lax.fori_loop(
        0, _K, step, jnp.zeros((_K, r), jnp.int32))


def _select_body(c4_ref, q_ref, sub_ref, w_ref, gidx_ref, scratch):
    qs = q_ref[...]
    for k in range(_K):
        xk = c4_ref[k, :, :]
        qk = qs[:, k:k + 1]
        ek = jnp.where(
            qk == 0, xk[:, 0:32],
            jnp.where(qk == 1, xk[:, 32:64],
                      jnp.where(qk == 2, xk[:, 64:96], xk[:, 96:128])))
        scratch[:, k * _SUB:(k + 1) * _SUB] = ek

    r, n = scratch.shape
    col = jax.lax.broadcasted_iota(jnp.int32, (r, n), 1)
    lane32 = jax.lax.broadcasted_iota(jnp.int32, (r, _K), 1)
    subs = sub_ref[...]

    def step(j, carry):
        vals, gidx = carry
        x = scratch[...]
        m = jnp.max(x, axis=1, keepdims=True)
        idx = jnp.min(jnp.where(x == m, col, n), axis=1, keepdims=True)
        sub_slot = idx // _SUB
        sub_val = jnp.sum(
            jnp.where(lane32 == sub_slot, subs, 0),
            axis=1, keepdims=True)
        vals = jnp.where(lane32 == j, m, vals)
        gidx = jnp.where(lane32 == j, sub_val * _SUB + idx % _SUB, gidx)
        scratch[...] = jnp.where(col == idx, _NEG, x)
        return vals, gidx

    vals, gidx = jax.lax.fori_loop(
        0, _K, step,
        (jnp.zeros((r, _K), jnp.float32), jnp.zeros((r, _K), jnp.int32)))
    gidx_ref[...] = gidx
    m = jnp.max(vals, axis=1, keepdims=True)
    e = jnp.exp(vals - m)
    w_ref[...] = e / jnp.sum(e, axis=1, keepdims=True)


_NC = 2
_NS = 16
_NW = _NC * _NS


def _sc_gather(table, idx2d, d):
    n_chunks = idx2d.shape[0]
    per_w = n_chunks // _NW
    mesh = plsc.VectorSubcoreMesh(core_axis_name="c", subcore_axis_name="s")

    @functools.partial(
        pl.kernel,
        out_type=jax.ShapeDtypeStruct((n_chunks * 128, d), jnp.float32),
        mesh=mesh,
        scratch_types=[
            pltpu.VMEM((per_w, 128), jnp.int32),
            pltpu.VMEM((128, d), jnp.float32),
            pltpu.SemaphoreType.DMA,
        ],
    )
    def k(table_hbm, idx_hbm, out_hbm, idx_v, rows_v, sem):
        wid = lax.axis_index("s") * _NC + lax.axis_index("c")
        pltpu.sync_copy(idx_hbm.at[pl.ds(wid * per_w, per_w)], idx_v)
        for j in range(per_w):
            pltpu.async_copy(table_hbm.at[idx_v.at[j]], rows_v, sem).wait()
            pltpu.sync_copy(
                rows_v, out_hbm.at[pl.ds((wid * per_w + j) * 128, 128)])

    return k(table, idx2d)


def _agg_body(g_ref, w_ref, wout_ref, o_ref):
    w = w_ref[...]
    acc = jnp.zeros((_RB, _D), jnp.float32)
    for k in range(_K):
        acc = acc + g_ref[:, k, :] * w[:, k:k + 1]
    o_ref[...] = jax.lax.dot_general(
        acc, wout_ref[...], (((1,), (1,)), ((), ())),
        preferred_element_type=jnp.float32)


@jax.jit
def kernel(query, pool, keys, W_out):
    q2 = query.reshape(_ROWS, _D)

    scores, maxes = pl.pallas_call(
        _score_body,
        grid=(_NCHUNK, _ROWS // _ARB),
        in_specs=[
            pl.BlockSpec((_ARB, _D), lambda c, r: (r, 0)),
            pl.BlockSpec((_CB, _D), lambda c, r: (c, 0)),
        ],
        out_specs=[
            pl.BlockSpec((_ARB, _CB // 128, 128), lambda c, r: (r, c, 0)),
            pl.BlockSpec((_CB // _SUB, _ARB), lambda c, r: (c, r)),
        ],
        out_shape=[
            jax.ShapeDtypeStruct((_ROWS, _NPAD // 128, 128), jnp.float32),
            jax.ShapeDtypeStruct((_NSUB, _ROWS), jnp.float32),
        ],
    )(q2, keys)

    sub_ids_t = pl.pallas_call(
        _topk_body,
        grid=(_ROWS // _RB,),
        in_specs=[pl.BlockSpec((_NSUB, _RB), lambda r: (0, r))],
        out_specs=pl.BlockSpec((_K, _RB), lambda r: (0, r)),
        out_shape=jax.ShapeDtypeStruct((_K, _ROWS), jnp.int32),
        scratch_shapes=[pltpu.VMEM((_NSUB, _RB), jnp.float32)],
    )(maxes)
    sub_ids = sub_ids_t.T

    row_lane = jnp.arange(_ROWS, dtype=jnp.int32)[None, :]
    tmat = (row_lane * (_NPAD // 128) + sub_ids_t // 4).reshape(-1, 128)
    cand4 = _sc_gather(scores.reshape(_ROWS * (_NPAD // 128), 128), tmat, 128)
    cand4 = cand4.reshape(_K, _ROWS, 128)
    quarters = sub_ids % 4

    weights, gidx = pl.pallas_call(
        _select_body,
        grid=(_ROWS // _RB,),
        in_specs=[
            pl.BlockSpec((_K, _RB, 128), lambda r: (0, r, 0)),
            pl.BlockSpec((_RB, _K), lambda r: (r, 0)),
            pl.BlockSpec((_RB, _K), lambda r: (r, 0)),
        ],
        out_specs=[
            pl.BlockSpec((_RB, _K), lambda r: (r, 0)),
            pl.BlockSpec((_RB, _K), lambda r: (r, 0)),
        ],
        out_shape=[
            jax.ShapeDtypeStruct((_ROWS, _K), jnp.float32),
            jax.ShapeDtypeStruct((_ROWS, _K), jnp.int32),
        ],
        scratch_shapes=[pltpu.VMEM((_RB, _K * _SUB), jnp.float32)],
    )(cand4, quarters, sub_ids)

    gathered = _sc_gather(pool, gidx.reshape(-1, 128), _D)
    gathered = gathered.reshape(_ROWS, _K, _D)

    out = pl.pallas_call(
        _agg_body,
        grid=(_ROWS // _RB,),
        in_specs=[
            pl.BlockSpec((_RB, _K, _D), lambda r: (r, 0, 0)),
            pl.BlockSpec((_RB, _K), lambda r: (r, 0)),
            pl.BlockSpec((_D, _D), lambda r: (0, 0)),
        ],
        out_specs=pl.BlockSpec((_RB, _D), lambda r: (r, 0)),
        out_shape=jax.ShapeDtypeStruct((_ROWS, _D), jnp.float32),
    )(gathered, weights, W_out)

    return out.reshape(query.shape[0], query.shape[1], _D)


# speedup vs baseline: 110.0404x; 1.0663x over previous
import functools

import jax
import jax.numpy as jnp
from jax import lax
from jax.experimental import pallas as pl
from jax.experimental.pallas import tpu as pltpu
from jax.experimental.pallas import tpu_sc as plsc

_POOL = 100000
_D = 256
_K = 32
_ROWS = 2048
_SUB = 32
_CB = 4096
_NCHUNK = 25
_NPAD = _CB * _NCHUNK
_NSUB = _NPAD // _SUB
_RB = 256
_NEG = float("-inf")


_ARB = 512
_NSUB_REAL = _POOL // _SUB


def _score_body(q_ref, k_ref, s_ref, m_ref):
    c = pl.program_id(0)
    s = jax.lax.dot_general(
        q_ref[...], k_ref[...], (((1,), (1,)), ((), ())),
        preferred_element_type=jnp.float32)
    s_ref[...] = s.reshape(_ARB, _CB // 128, 128)
    st = jax.lax.dot_general(
        k_ref[...], q_ref[...], (((1,), (1,)), ((), ())),
        preferred_element_type=jnp.float32)
    m = jnp.max(st.reshape(_CB // _SUB, _SUB, _ARB), axis=1)
    sub = jax.lax.broadcasted_iota(jnp.int32, m.shape, 0) + c * (_CB // _SUB)
    m_ref[...] = jnp.where(sub < _NSUB_REAL, m, _NEG)


def _topk_body(x_ref, idx_ref, scratch):
    scratch[...] = x_ref[...]
    n, r = x_ref.shape
    row = jax.lax.broadcasted_iota(jnp.int32, (n, r), 0)
    slot = jax.lax.broadcasted_iota(jnp.int32, (_K, r), 0)

    def step(j, idxs):
        x = scratch[...]
        m = jnp.max(x, axis=0, keepdims=True)
        idx = jnp.min(jnp.where(x == m, row, n), axis=0, keepdims=True)
        idxs = jnp.where(slot == j, idx, idxs)
        scratch[...] = jnp.where(row == idx, _NEG, x)
        return idxs

    idx_ref[...] = jax.lax.fori_loop(
        0, _K, step, jnp.zeros((_K, r), jnp.int32))


def _select_body(c4_ref, q_ref, sub_ref, w_ref, gidx_ref, scratch):
    qs = q_ref[...]
    for k in range(_K):
        xk = c4_ref[k, :, :]
        qk = qs[:, k:k + 1]
        ek = jnp.where(
            qk == 0, xk[:, 0:32],
            jnp.where(qk == 1, xk[:, 32:64],
                      jnp.where(qk == 2, xk[:, 64:96], xk[:, 96:128])))
        scratch[:, k * _SUB:(k + 1) * _SUB] = ek

    r, n = scratch.shape
    col = jax.lax.broadcasted_iota(jnp.int32, (r, n), 1)
    lane32 = jax.lax.broadcasted_iota(jnp.int32, (r, _K), 1)
    subs = sub_ref[...]

    def step(j, carry):
        vals, gidx = carry
        x = scratch[...]
        m = jnp.max(x, axis=1, keepdims=True)
        idx = jnp.min(jnp.where(x == m, col, n), axis=1, keepdims=True)
        sub_slot = idx // _SUB
        sub_val = jnp.sum(
            jnp.where(lane32 == sub_slot, subs, 0),
            axis=1, keepdims=True)
        vals = jnp.where(lane32 == j, m, vals)
        gidx = jnp.where(lane32 == j, sub_val * _SUB + idx % _SUB, gidx)
        scratch[...] = jnp.where(col == idx, _NEG, x)
        return vals, gidx

    vals, gidx = jax.lax.fori_loop(
        0, _K, step,
        (jnp.zeros((r, _K), jnp.float32), jnp.zeros((r, _K), jnp.int32)))
    gidx_ref[...] = gidx
    m = jnp.max(vals, axis=1, keepdims=True)
    e = jnp.exp(vals - m)
    w_ref[...] = e / jnp.sum(e, axis=1, keepdims=True)


_NC = 2
_NS = 16
_NW = _NC * _NS


def _sc_gather(table, idx2d, d):
    n_chunks = idx2d.shape[0]
    per_w = n_chunks // _NW
    mesh = plsc.VectorSubcoreMesh(core_axis_name="c", subcore_axis_name="s")

    @functools.partial(
        pl.kernel,
        out_type=jax.ShapeDtypeStruct((n_chunks * 128, d), jnp.float32),
        mesh=mesh,
        scratch_types=[
            pltpu.VMEM((per_w, 128), jnp.int32),
            pltpu.VMEM((128, d), jnp.float32),
            pltpu.SemaphoreType.DMA,
        ],
    )
    def k(table_hbm, idx_hbm, out_hbm, idx_v, rows_v, sem):
        wid = lax.axis_index("s") * _NC + lax.axis_index("c")
        pltpu.sync_copy(idx_hbm.at[pl.ds(wid * per_w, per_w)], idx_v)
        for j in range(per_w):
            pltpu.async_copy(table_hbm.at[idx_v.at[j]], rows_v, sem).wait()
            pltpu.sync_copy(
                rows_v, out_hbm.at[pl.ds((wid * per_w + j) * 128, 128)])

    return k(table, idx2d)


def _agg_body(g_ref, w_ref, wout_ref, o_ref):
    w = w_ref[...]
    acc = jnp.zeros((_RB, _D), jnp.float32)
    for k in range(_K):
        acc = acc + g_ref[:, k, :] * w[:, k:k + 1]
    o_ref[...] = jax.lax.dot_general(
        acc, wout_ref[...], (((1,), (1,)), ((), ())),
        preferred_element_type=jnp.float32)


@jax.jit
def kernel(query, pool, keys, W_out):
    q2 = query.reshape(_ROWS, _D)

    scores, maxes = pl.pallas_call(
        _score_body,
        grid=(_NCHUNK, _ROWS // _ARB),
        in_specs=[
            pl.BlockSpec((_ARB, _D), lambda c, r: (r, 0)),
            pl.BlockSpec((_CB, _D), lambda c, r: (c, 0)),
        ],
        out_specs=[
            pl.BlockSpec((_ARB, _CB // 128, 128), lambda c, r: (r, c, 0)),
            pl.BlockSpec((_CB // _SUB, _ARB), lambda c, r: (c, r)),
        ],
        out_shape=[
            jax.ShapeDtypeStruct((_ROWS, _NPAD // 128, 128), jnp.float32),
            jax.ShapeDtypeStruct((_NSUB, _ROWS), jnp.float32),
        ],
    )(q2, keys)

    sub_ids_t = pl.pallas_call(
        _topk_body,
        grid=(_ROWS // _RB,),
        in_specs=[pl.BlockSpec((_NSUB, _RB), lambda r: (0, r))],
        out_specs=pl.BlockSpec((_K, _RB), lambda r: (0, r)),
        out_shape=jax.ShapeDtypeStruct((_K, _ROWS), jnp.int32),
        scratch_shapes=[pltpu.VMEM((_NSUB, _RB), jnp.float32)],
    )(maxes)
    sub_ids = sub_ids_t.T

    row_lane = jnp.arange(_ROWS, dtype=jnp.int32)[None, :]
    tmat = (row_lane * (_NPAD // 128) + sub_ids_t // 4).reshape(-1, 128)
    cand4 = _sc_gather(scores.reshape(_ROWS * (_NPAD // 128), 128), tmat, 128)
    cand4 = cand4.reshape(_K, _ROWS, 128)
    quarters = sub_ids % 4

    weights, gidx = pl.pallas_call(
        _select_body,
        grid=(_ROWS // _RB,),
        in_specs=[
            pl.BlockSpec((_K, _RB, 128), lambda r: (0, r, 0)),
            pl.BlockSpec((_RB, _K), lambda r: (r, 0)),
            pl.BlockSpec((_RB, _K), lambda r: (r, 0)),
        ],
        out_specs=[
            pl.BlockSpec((_RB, _K), lambda r: (r, 0)),
            pl.BlockSpec((_RB, _K), lambda r: (r, 0)),
        ],
        out_shape=[
            jax.ShapeDtypeStruct((_ROWS, _K), jnp.float32),
            jax.ShapeDtypeStruct((_ROWS, _K), jnp.int32),
        ],
        scratch_shapes=[pltpu.VMEM((_RB, _K * _SUB), jnp.float32)],
    )(cand4, quarters, sub_ids)

    gathered = _sc_gather(pool, gidx.reshape(-1, 128), _D)
    gathered = gathered.reshape(_ROWS, _K, _D)

    out = pl.pallas_call(
        _agg_body,
        grid=(_ROWS // _RB,),
        in_specs=[
            pl.BlockSpec((_RB, _K, _D), lambda r: (r, 0, 0)),
            pl.BlockSpec((_RB, _K), lambda r: (r, 0)),
            pl.BlockSpec((_D, _D), lambda r: (0, 0)),
        ],
        out_specs=pl.BlockSpec((_RB, _D), lambda r: (r, 0)),
        out_shape=jax.ShapeDtypeStruct((_ROWS, _D), jnp.float32),
    )(gathered, weights, W_out)

    return out.reshape(query.shape[0], query.shape[1], _D)


# speedup vs baseline: 112.7167x; 1.0243x over previous
import functools

import jax
import jax.numpy as jnp
from jax import lax
from jax.experimental import pallas as pl
from jax.experimental.pallas import tpu as pltpu
from jax.experimental.pallas import tpu_sc as plsc

_POOL = 100000
_D = 256
_K = 32
_ROWS = 2048
_SUB = 32
_CB = 4096
_NCHUNK = 25
_NPAD = _CB * _NCHUNK
_NSUB = _NPAD // _SUB
_RB = 256
_NEG = float("-inf")


_ARB = 512
_NSUB_REAL = _POOL // _SUB


def _score_body(q_ref, k_ref, s_ref, m_ref):
    c = pl.program_id(0)
    s = jax.lax.dot_general(
        q_ref[...], k_ref[...], (((1,), (1,)), ((), ())),
        preferred_element_type=jnp.float32)
    s_ref[...] = s.reshape(_ARB, _CB // 128, 128)
    st = jax.lax.dot_general(
        k_ref[...], q_ref[...], (((1,), (1,)), ((), ())),
        preferred_element_type=jnp.float32)
    m = jnp.max(st.reshape(_CB // _SUB, _SUB, _ARB), axis=1)
    sub = jax.lax.broadcasted_iota(jnp.int32, m.shape, 0) + c * (_CB // _SUB)
    m_ref[...] = jnp.where(sub < _NSUB_REAL, m, _NEG)


def _topk_body(x_ref, idx_ref, scratch):
    scratch[...] = x_ref[...]
    n, r = x_ref.shape
    row = jax.lax.broadcasted_iota(jnp.int32, (n, r), 0)
    slot = jax.lax.broadcasted_iota(jnp.int32, (_K, r), 0)

    def step(j, idxs):
        x = scratch[...]
        m = jnp.max(x, axis=0, keepdims=True)
        idx = jnp.min(jnp.where(x == m, row, n), axis=0, keepdims=True)
        idxs = jnp.where(slot == j, idx, idxs)
        scratch[...] = jnp.where(row == idx, _NEG, x)
        return idxs

    idx_ref[...] = jax.lax.fori_loop(
        0, _K, step, jnp.zeros((_K, r), jnp.int32))


def _select_body(c4_ref, q_ref, sub_ref, w_ref, gidx_ref, scratch):
    qs = q_ref[...]
    for k in range(_K):
        xk = c4_ref[k, :, :]
        qk = qs[:, k:k + 1]
        ek = jnp.where(
            qk == 0, xk[:, 0:32],
            jnp.where(qk == 1, xk[:, 32:64],
                      jnp.where(qk == 2, xk[:, 64:96], xk[:, 96:128])))
        scratch[:, k * _SUB:(k + 1) * _SUB] = ek

    r, n = scratch.shape
    col = jax.lax.broadcasted_iota(jnp.int32, (r, n), 1)
    lane32 = jax.lax.broadcasted_iota(jnp.int32, (r, _K), 1)
    subs = sub_ref[...]

    def step(j, carry):
        vals, gidx = carry
        x = scratch[...]
        m = jnp.max(x, axis=1, keepdims=True)
        idx = jnp.min(jnp.where(x == m, col, n), axis=1, keepdims=True)
        sub_slot = idx // _SUB
        sub_val = jnp.sum(
            jnp.where(lane32 == sub_slot, subs, 0),
            axis=1, keepdims=True)
        vals = jnp.where(lane32 == j, m, vals)
        gidx = jnp.where(lane32 == j, sub_val * _SUB + idx % _SUB, gidx)
        scratch[...] = jnp.where(col == idx, _NEG, x)
        return vals, gidx

    vals, gidx = jax.lax.fori_loop(
        0, _K, step,
        (jnp.zeros((r, _K), jnp.float32), jnp.zeros((r, _K), jnp.int32)))
    gidx_ref[...] = gidx
    m = jnp.max(vals, axis=1, keepdims=True)
    e = jnp.exp(vals - m)
    w_ref[...] = e / jnp.sum(e, axis=1, keepdims=True)


_NC = 2
_NS = 16
_NW = _NC * _NS


def _sc_gather(table, idx2d, d):
    n_chunks = idx2d.shape[0]
    per_w = n_chunks // _NW
    mesh = plsc.VectorSubcoreMesh(core_axis_name="c", subcore_axis_name="s")

    @functools.partial(
        pl.kernel,
        out_type=jax.ShapeDtypeStruct((n_chunks * 128, d), jnp.float32),
        mesh=mesh,
        scratch_types=[
            pltpu.VMEM((per_w, 128), jnp.int32),
            pltpu.VMEM((128, d), jnp.float32),
            pltpu.SemaphoreType.DMA,
        ],
    )
    def k(table_hbm, idx_hbm, out_hbm, idx_v, rows_v, sem):
        wid = lax.axis_index("s") * _NC + lax.axis_index("c")
        pltpu.sync_copy(idx_hbm.at[pl.ds(wid * per_w, per_w)], idx_v)
        for j in range(per_w):
            pltpu.async_copy(table_hbm.at[idx_v.at[j]], rows_v, sem).wait()
            pltpu.sync_copy(
                rows_v, out_hbm.at[pl.ds((wid * per_w + j) * 128, 128)])

    return k(table, idx2d)


def _agg_body(g_ref, w_ref, wout_ref, o_ref):
    acc = jnp.sum(g_ref[...] * w_ref[...][:, :, None], axis=1)
    o_ref[...] = jax.lax.dot_general(
        acc, wout_ref[...], (((1,), (1,)), ((), ())),
        preferred_element_type=jnp.float32)


@jax.jit
def kernel(query, pool, keys, W_out):
    q2 = query.reshape(_ROWS, _D)

    scores, maxes = pl.pallas_call(
        _score_body,
        grid=(_NCHUNK, _ROWS // _ARB),
        in_specs=[
            pl.BlockSpec((_ARB, _D), lambda c, r: (r, 0)),
            pl.BlockSpec((_CB, _D), lambda c, r: (c, 0)),
        ],
        out_specs=[
            pl.BlockSpec((_ARB, _CB // 128, 128), lambda c, r: (r, c, 0)),
            pl.BlockSpec((_CB // _SUB, _ARB), lambda c, r: (c, r)),
        ],
        out_shape=[
            jax.ShapeDtypeStruct((_ROWS, _NPAD // 128, 128), jnp.float32),
            jax.ShapeDtypeStruct((_NSUB, _ROWS), jnp.float32),
        ],
    )(q2, keys)

    sub_ids_t = pl.pallas_call(
        _topk_body,
        grid=(_ROWS // _RB,),
        in_specs=[pl.BlockSpec((_NSUB, _RB), lambda r: (0, r))],
        out_specs=pl.BlockSpec((_K, _RB), lambda r: (0, r)),
        out_shape=jax.ShapeDtypeStruct((_K, _ROWS), jnp.int32),
        scratch_shapes=[pltpu.VMEM((_NSUB, _RB), jnp.float32)],
    )(maxes)
    sub_ids = sub_ids_t.T

    row_lane = jnp.arange(_ROWS, dtype=jnp.int32)[None, :]
    tmat = (row_lane * (_NPAD // 128) + sub_ids_t // 4).reshape(-1, 128)
    cand4 = _sc_gather(scores.reshape(_ROWS * (_NPAD // 128), 128), tmat, 128)
    cand4 = cand4.reshape(_K, _ROWS, 128)
    quarters = sub_ids % 4

    weights, gidx = pl.pallas_call(
        _select_body,
        grid=(_ROWS // _RB,),
        in_specs=[
            pl.BlockSpec((_K, _RB, 128), lambda r: (0, r, 0)),
            pl.BlockSpec((_RB, _K), lambda r: (r, 0)),
            pl.BlockSpec((_RB, _K), lambda r: (r, 0)),
        ],
        out_specs=[
            pl.BlockSpec((_RB, _K), lambda r: (r, 0)),
            pl.BlockSpec((_RB, _K), lambda r: (r, 0)),
        ],
        out_shape=[
            jax.ShapeDtypeStruct((_ROWS, _K), jnp.float32),
            jax.ShapeDtypeStruct((_ROWS, _K), jnp.int32),
        ],
        scratch_shapes=[pltpu.VMEM((_RB, _K * _SUB), jnp.float32)],
    )(cand4, quarters, sub_ids)

    gathered = _sc_gather(pool, gidx.reshape(-1, 128), _D)
    gathered = gathered.reshape(_ROWS, _K, _D)

    out = pl.pallas_call(
        _agg_body,
        grid=(_ROWS // _RB,),
        in_specs=[
            pl.BlockSpec((_RB, _K, _D), lambda r: (r, 0, 0)),
            pl.BlockSpec((_RB, _K), lambda r: (r, 0)),
            pl.BlockSpec((_D, _D), lambda r: (0, 0)),
        ],
        out_specs=pl.BlockSpec((_RB, _D), lambda r: (r, 0)),
        out_shape=jax.ShapeDtypeStruct((_ROWS, _D), jnp.float32),
    )(gathered, weights, W_out)

    return out.reshape(query.shape[0], query.shape[1], _D)


# speedup vs baseline: 117.0264x; 1.0382x over previous
import functools

import jax
import jax.numpy as jnp
from jax import lax
from jax.experimental import pallas as pl
from jax.experimental.pallas import tpu as pltpu
from jax.experimental.pallas import tpu_sc as plsc

_POOL = 100000
_D = 256
_K = 32
_ROWS = 2048
_SUB = 32
_CB = 4096
_NCHUNK = 25
_NPAD = _CB * _NCHUNK
_NSUB = _NPAD // _SUB
_RB = 512
_NEG = float("-inf")


_ARB = 512
_NSUB_REAL = _POOL // _SUB


def _score_body(q_ref, k_ref, s_ref, m_ref):
    c = pl.program_id(0)
    s = jax.lax.dot_general(
        q_ref[...], k_ref[...], (((1,), (1,)), ((), ())),
        preferred_element_type=jnp.float32)
    s_ref[...] = s.reshape(_ARB, _CB // 128, 128)
    st = jax.lax.dot_general(
        k_ref[...], q_ref[...], (((1,), (1,)), ((), ())),
        preferred_element_type=jnp.float32)
    m = jnp.max(st.reshape(_CB // _SUB, _SUB, _ARB), axis=1)
    sub = jax.lax.broadcasted_iota(jnp.int32, m.shape, 0) + c * (_CB // _SUB)
    m_ref[...] = jnp.where(sub < _NSUB_REAL, m, _NEG)


def _topk_body(x_ref, idx_ref, scratch):
    scratch[...] = x_ref[...]
    n, r = x_ref.shape
    row = jax.lax.broadcasted_iota(jnp.int32, (n, r), 0)
    slot = jax.lax.broadcasted_iota(jnp.int32, (_K, r), 0)

    def step(j, idxs):
        x = scratch[...]
        m = jnp.max(x, axis=0, keepdims=True)
        idx = jnp.min(jnp.where(x == m, row, n), axis=0, keepdims=True)
        idxs = jnp.where(slot == j, idx, idxs)
        scratch[...] = jnp.where(row == idx, _NEG, x)
        return idxs

    idx_ref[...] = jax.lax.fori_loop(
        0, _K, step, jnp.zeros((_K, r), jnp.int32))


def _select_body(c4_ref, q_ref, sub_ref, w_ref, gidx_ref, scratch):
    qs = q_ref[...]
    for k in range(_K):
        xk = c4_ref[k, :, :]
        qk = qs[:, k:k + 1]
        ek = jnp.where(
            qk == 0, xk[:, 0:32],
            jnp.where(qk == 1, xk[:, 32:64],
                      jnp.where(qk == 2, xk[:, 64:96], xk[:, 96:128])))
        scratch[:, k * _SUB:(k + 1) * _SUB] = ek

    r, n = scratch.shape
    col = jax.lax.broadcasted_iota(jnp.int32, (r, n), 1)
    lane32 = jax.lax.broadcasted_iota(jnp.int32, (r, _K), 1)
    subs = sub_ref[...]

    def step(j, carry):
        vals, gidx = carry
        x = scratch[...]
        m = jnp.max(x, axis=1, keepdims=True)
        idx = jnp.min(jnp.where(x == m, col, n), axis=1, keepdims=True)
        sub_slot = idx // _SUB
        sub_val = jnp.sum(
            jnp.where(lane32 == sub_slot, subs, 0),
            axis=1, keepdims=True)
        vals = jnp.where(lane32 == j, m, vals)
        gidx = jnp.where(lane32 == j, sub_val * _SUB + idx % _SUB, gidx)
        scratch[...] = jnp.where(col == idx, _NEG, x)
        return vals, gidx

    vals, gidx = jax.lax.fori_loop(
        0, _K, step,
        (jnp.zeros((r, _K), jnp.float32), jnp.zeros((r, _K), jnp.int32)))
    gidx_ref[...] = gidx
    m = jnp.max(vals, axis=1, keepdims=True)
    e = jnp.exp(vals - m)
    w_ref[...] = e / jnp.sum(e, axis=1, keepdims=True)


_NC = 2
_NS = 16
_NW = _NC * _NS


def _sc_gather(table, idx2d, d):
    n_chunks = idx2d.shape[0]
    per_w = n_chunks // _NW
    mesh = plsc.VectorSubcoreMesh(core_axis_name="c", subcore_axis_name="s")

    @functools.partial(
        pl.kernel,
        out_type=jax.ShapeDtypeStruct((n_chunks * 128, d), jnp.float32),
        mesh=mesh,
        scratch_types=[
            pltpu.VMEM((per_w, 128), jnp.int32),
            pltpu.VMEM((128, d), jnp.float32),
            pltpu.SemaphoreType.DMA,
        ],
    )
    def k(table_hbm, idx_hbm, out_hbm, idx_v, rows_v, sem):
        wid = lax.axis_index("s") * _NC + lax.axis_index("c")
        pltpu.sync_copy(idx_hbm.at[pl.ds(wid * per_w, per_w)], idx_v)
        for j in range(per_w):
            pltpu.async_copy(table_hbm.at[idx_v.at[j]], rows_v, sem).wait()
            pltpu.sync_copy(
                rows_v, out_hbm.at[pl.ds((wid * per_w + j) * 128, 128)])

    return k(table, idx2d)


def _agg_body(g_ref, w_ref, wout_ref, o_ref):
    acc = jnp.sum(g_ref[...] * w_ref[...][:, :, None], axis=1)
    o_ref[...] = jax.lax.dot_general(
        acc, wout_ref[...], (((1,), (1,)), ((), ())),
        preferred_element_type=jnp.float32)


@jax.jit
def kernel(query, pool, keys, W_out):
    q2 = query.reshape(_ROWS, _D)

    scores, maxes = pl.pallas_call(
        _score_body,
        grid=(_NCHUNK, _ROWS // _ARB),
        in_specs=[
            pl.BlockSpec((_ARB, _D), lambda c, r: (r, 0)),
            pl.BlockSpec((_CB, _D), lambda c, r: (c, 0)),
        ],
        out_specs=[
            pl.BlockSpec((_ARB, _CB // 128, 128), lambda c, r: (r, c, 0)),
            pl.BlockSpec((_CB // _SUB, _ARB), lambda c, r: (c, r)),
        ],
        out_shape=[
            jax.ShapeDtypeStruct((_ROWS, _NPAD // 128, 128), jnp.float32),
            jax.ShapeDtypeStruct((_NSUB, _ROWS), jnp.float32),
        ],
    )(q2, keys)

    sub_ids_t = pl.pallas_call(
        _topk_body,
        grid=(_ROWS // _RB,),
        in_specs=[pl.BlockSpec((_NSUB, _RB), lambda r: (0, r))],
        out_specs=pl.BlockSpec((_K, _RB), lambda r: (0, r)),
        out_shape=jax.ShapeDtypeStruct((_K, _ROWS), jnp.int32),
        scratch_shapes=[pltpu.VMEM((_NSUB, _RB), jnp.float32)],
    )(maxes)
    sub_ids = sub_ids_t.T

    row_lane = jnp.arange(_ROWS, dtype=jnp.int32)[None, :]
    tmat = (row_lane * (_NPAD // 128) + sub_ids_t // 4).reshape(-1, 128)
    cand4 = _sc_gather(scores.reshape(_ROWS * (_NPAD // 128), 128), tmat, 128)
    cand4 = cand4.reshape(_K, _ROWS, 128)
    quarters = sub_ids % 4

    weights, gidx = pl.pallas_call(
        _select_body,
        grid=(_ROWS // _RB,),
        in_specs=[
            pl.BlockSpec((_K, _RB, 128), lambda r: (0, r, 0)),
            pl.BlockSpec((_RB, _K), lambda r: (r, 0)),
            pl.BlockSpec((_RB, _K), lambda r: (r, 0)),
        ],
        out_specs=[
            pl.BlockSpec((_RB, _K), lambda r: (r, 0)),
            pl.BlockSpec((_RB, _K), lambda r: (r, 0)),
        ],
        out_shape=[
            jax.ShapeDtypeStruct((_ROWS, _K), jnp.float32),
            jax.ShapeDtypeStruct((_ROWS, _K), jnp.int32),
        ],
        scratch_shapes=[pltpu.VMEM((_RB, _K * _SUB), jnp.float32)],
    )(cand4, quarters, sub_ids)

    gathered = _sc_gather(pool, gidx.reshape(-1, 128), _D)
    gathered = gathered.reshape(_ROWS, _K, _D)

    out = pl.pallas_call(
        _agg_body,
        grid=(_ROWS // _RB,),
        in_specs=[
            pl.BlockSpec((_RB, _K, _D), lambda r: (r, 0, 0)),
            pl.BlockSpec((_RB, _K), lambda r: (r, 0)),
            pl.BlockSpec((_D, _D), lambda r: (0, 0)),
        ],
        out_specs=pl.BlockSpec((_RB, _D), lambda r: (r, 0)),
        out_shape=jax.ShapeDtypeStruct((_ROWS, _D), jnp.float32),
    )(gathered, weights, W_out)

    return out.reshape(query.shape[0], query.shape[1], _D)


# speedup vs baseline: 129.0785x; 1.1030x over previous
import functools

import jax
import jax.numpy as jnp
from jax import lax
from jax.experimental import pallas as pl
from jax.experimental.pallas import tpu as pltpu
from jax.experimental.pallas import tpu_sc as plsc

_POOL = 100000
_D = 256
_K = 32
_ROWS = 2048
_SUB = 32
_CB = 4096
_NCHUNK = 25
_NPAD = _CB * _NCHUNK
_NSUB = _NPAD // _SUB
_RB = 512
_NEG = float("-inf")


_ARB = 512
_NSUB_REAL = _POOL // _SUB


def _score_body(q_ref, k_ref, s_ref, m_ref):
    c = pl.program_id(0)
    s = jax.lax.dot_general(
        q_ref[...], k_ref[...], (((1,), (1,)), ((), ())),
        preferred_element_type=jnp.float32)
    s_ref[...] = s.reshape(_ARB, _CB // 128, 128)
    st = jax.lax.dot_general(
        k_ref[...], q_ref[...], (((1,), (1,)), ((), ())),
        preferred_element_type=jnp.float32)
    m = jnp.max(st.reshape(_CB // _SUB, _SUB, _ARB), axis=1)
    sub = jax.lax.broadcasted_iota(jnp.int32, m.shape, 0) + c * (_CB // _SUB)
    m_ref[...] = jnp.where(sub < _NSUB_REAL, m, _NEG)


def _topk_body(x_ref, idx_ref, scratch):
    scratch[...] = x_ref[...]
    n, r = x_ref.shape
    row = jax.lax.broadcasted_iota(jnp.int32, (n, r), 0)
    slot = jax.lax.broadcasted_iota(jnp.int32, (_K, r), 0)

    def step(j, idxs):
        x = scratch[...]
        idx = jnp.argmax(x, axis=0, keepdims=True).astype(jnp.int32)
        idxs = jnp.where(slot == j, idx, idxs)
        scratch[...] = jnp.where(row == idx, _NEG, x)
        return idxs

    idx_ref[...] = jax.lax.fori_loop(
        0, _K, step, jnp.zeros((_K, r), jnp.int32))


def _select_body(c4_ref, q_ref, sub_ref, w_ref, gidx_ref, scratch):
    qs = q_ref[...]
    for k in range(_K):
        xk = c4_ref[k, :, :]
        qk = qs[:, k:k + 1]
        ek = jnp.where(
            qk == 0, xk[:, 0:32],
            jnp.where(qk == 1, xk[:, 32:64],
                      jnp.where(qk == 2, xk[:, 64:96], xk[:, 96:128])))
        scratch[:, k * _SUB:(k + 1) * _SUB] = ek

    r, n = scratch.shape
    col = jax.lax.broadcasted_iota(jnp.int32, (r, n), 1)
    lane32 = jax.lax.broadcasted_iota(jnp.int32, (r, _K), 1)
    subs = sub_ref[...]

    def step(j, carry):
        vals, gidx = carry
        x = scratch[...]
        m = jnp.max(x, axis=1, keepdims=True)
        idx = jnp.min(jnp.where(x == m, col, n), axis=1, keepdims=True)
        sub_slot = idx // _SUB
        sub_val = jnp.sum(
            jnp.where(lane32 == sub_slot, subs, 0),
            axis=1, keepdims=True)
        vals = jnp.where(lane32 == j, m, vals)
        gidx = jnp.where(lane32 == j, sub_val * _SUB + idx % _SUB, gidx)
        scratch[...] = jnp.where(col == idx, _NEG, x)
        return vals, gidx

    vals, gidx = jax.lax.fori_loop(
        0, _K, step,
        (jnp.zeros((r, _K), jnp.float32), jnp.zeros((r, _K), jnp.int32)))
    gidx_ref[...] = gidx
    m = jnp.max(vals, axis=1, keepdims=True)
    e = jnp.exp(vals - m)
    w_ref[...] = e / jnp.sum(e, axis=1, keepdims=True)


_NC = 2
_NS = 16
_NW = _NC * _NS


def _sc_gather(table, idx2d, d):
    n_chunks = idx2d.shape[0]
    per_w = n_chunks // _NW
    mesh = plsc.VectorSubcoreMesh(core_axis_name="c", subcore_axis_name="s")

    @functools.partial(
        pl.kernel,
        out_type=jax.ShapeDtypeStruct((n_chunks * 128, d), jnp.float32),
        mesh=mesh,
        scratch_types=[
            pltpu.VMEM((per_w, 128), jnp.int32),
            pltpu.VMEM((128, d), jnp.float32),
            pltpu.SemaphoreType.DMA,
        ],
    )
    def k(table_hbm, idx_hbm, out_hbm, idx_v, rows_v, sem):
        wid = lax.axis_index("s") * _NC + lax.axis_index("c")
        pltpu.sync_copy(idx_hbm.at[pl.ds(wid * per_w, per_w)], idx_v)
        for j in range(per_w):
            pltpu.async_copy(table_hbm.at[idx_v.at[j]], rows_v, sem).wait()
            pltpu.sync_copy(
                rows_v, out_hbm.at[pl.ds((wid * per_w + j) * 128, 128)])

    return k(table, idx2d)


def _agg_body(g_ref, w_ref, wout_ref, o_ref):
    acc = jnp.sum(g_ref[...] * w_ref[...][:, :, None], axis=1)
    o_ref[...] = jax.lax.dot_general(
        acc, wout_ref[...], (((1,), (1,)), ((), ())),
        preferred_element_type=jnp.float32)


@jax.jit
def kernel(query, pool, keys, W_out):
    q2 = query.reshape(_ROWS, _D)

    scores, maxes = pl.pallas_call(
        _score_body,
        grid=(_NCHUNK, _ROWS // _ARB),
        in_specs=[
            pl.BlockSpec((_ARB, _D), lambda c, r: (r, 0)),
            pl.BlockSpec((_CB, _D), lambda c, r: (c, 0)),
        ],
        out_specs=[
            pl.BlockSpec((_ARB, _CB // 128, 128), lambda c, r: (r, c, 0)),
            pl.BlockSpec((_CB // _SUB, _ARB), lambda c, r: (c, r)),
        ],
        out_shape=[
            jax.ShapeDtypeStruct((_ROWS, _NPAD // 128, 128), jnp.float32),
            jax.ShapeDtypeStruct((_NSUB, _ROWS), jnp.float32),
        ],
    )(q2, keys)

    sub_ids_t = pl.pallas_call(
        _topk_body,
        grid=(_ROWS // _RB,),
        in_specs=[pl.BlockSpec((_NSUB, _RB), lambda r: (0, r))],
        out_specs=pl.BlockSpec((_K, _RB), lambda r: (0, r)),
        out_shape=jax.ShapeDtypeStruct((_K, _ROWS), jnp.int32),
        scratch_shapes=[pltpu.VMEM((_NSUB, _RB), jnp.float32)],
    )(maxes)
    sub_ids = sub_ids_t.T

    row_lane = jnp.arange(_ROWS, dtype=jnp.int32)[None, :]
    tmat = (row_lane * (_NPAD // 128) + sub_ids_t // 4).reshape(-1, 128)
    cand4 = _sc_gather(scores.reshape(_ROWS * (_NPAD // 128), 128), tmat, 128)
    cand4 = cand4.reshape(_K, _ROWS, 128)
    quarters = sub_ids % 4

    weights, gidx = pl.pallas_call(
        _select_body,
        grid=(_ROWS // _RB,),
        in_specs=[
            pl.BlockSpec((_K, _RB, 128), lambda r: (0, r, 0)),
            pl.BlockSpec((_RB, _K), lambda r: (r, 0)),
            pl.BlockSpec((_RB, _K), lambda r: (r, 0)),
        ],
        out_specs=[
            pl.BlockSpec((_RB, _K), lambda r: (r, 0)),
            pl.BlockSpec((_RB, _K), lambda r: (r, 0)),
        ],
        out_shape=[
            jax.ShapeDtypeStruct((_ROWS, _K), jnp.float32),
            jax.ShapeDtypeStruct((_ROWS, _K), jnp.int32),
        ],
        scratch_shapes=[pltpu.VMEM((_RB, _K * _SUB), jnp.float32)],
    )(cand4, quarters, sub_ids)

    gathered = _sc_gather(pool, gidx.reshape(-1, 128), _D)
    gathered = gathered.reshape(_ROWS, _K, _D)

    out = pl.pallas_call(
        _agg_body,
        grid=(_ROWS // _RB,),
        in_specs=[
            pl.BlockSpec((_RB, _K, _D), lambda r: (r, 0, 0)),
            pl.BlockSpec((_RB, _K), lambda r: (r, 0)),
            pl.BlockSpec((_D, _D), lambda r: (0, 0)),
        ],
        out_specs=pl.BlockSpec((_RB, _D), lambda r: (r, 0)),
        out_shape=jax.ShapeDtypeStruct((_ROWS, _D), jnp.float32),
    )(gathered, weights, W_out)

    return out.reshape(query.shape[0], query.shape[1], _D)
